# baseline (speedup 1.0000x reference)
import functools

import jax
import jax.numpy as jnp
from jax import lax
from jax.experimental import pallas as pl
from jax.experimental.pallas import tpu as pltpu
from jax.experimental.pallas import tpu_sc as plsc

N_SIDE = 5000
NP = 6144
D = 128
E = 160000
NT = 16
EPT = E // NT
EC = 80
NCH = EPT // EC
EC2 = 112
NCH2 = -(-EPT // EC2)
RPT = NP // NT
RB = 768


def _f32(shape):
    return jax.ShapeDtypeStruct(shape, jnp.float32)


@functools.partial(
    pl.kernel,
    out_type=(_f32((2 * NP,)), _f32((2 * NP,))),
    mesh=plsc.VectorSubcoreMesh(core_axis_name="c", subcore_axis_name="s"),
    scratch_types=[
        pltpu.VMEM((NCH, EC), jnp.int32),
        pltpu.VMEM((EC,), jnp.float32),
        pltpu.VMEM((RPT,), jnp.float32),
        pltpu.VMEM((RPT,), jnp.float32),
        pltpu.VMEM((RPT,), jnp.float32),
        pltpu.VMEM_SHARED((NP,), jnp.float32),
    ],
)
def _sc_stats(idx_all, ones1, zeros1,
              dinv, dinv2,
              idx_v, ones_v, degbuf, dvb, dv2b, degacc):
    tid = lax.axis_index("s")
    cid = lax.axis_index("c")
    r0 = tid * RPT

    pltpu.sync_copy(zeros1, degbuf)
    pltpu.sync_copy(degbuf, degacc.at[pl.ds(r0, RPT)])
    pltpu.sync_copy(ones1, ones_v)
    pltpu.sync_copy(idx_all.at[cid, tid], idx_v)

    plsc.subcore_barrier()

    def scat(c, carry):
        pltpu.sync_copy(ones_v, degacc.at[idx_v.at[c]], add=True)
        return carry

    lax.fori_loop(0, NCH, scat, 0)

    plsc.subcore_barrier()

    pltpu.sync_copy(degacc.at[pl.ds(r0, RPT)], degbuf)

    def conv(k, carry):
        deg = degbuf[pl.ds(k * 16, 16)]
        bits = lax.bitcast_convert_type(deg, jnp.int32)
        bits = jnp.int32(0x5F3759DF) - (bits >> 1)
        y = lax.bitcast_convert_type(bits, jnp.float32)
        for _ in range(3):
            y = y * (1.5 - 0.5 * deg * y * y)
        y = jnp.where(deg >= 0.5, y, 0.0)
        dvb[pl.ds(k * 16, 16)] = y
        dv2b[pl.ds(k * 16, 16)] = y * y
        return carry

    lax.fori_loop(0, RPT // 16, conv, 0)

    out0 = cid * NP + r0
    pltpu.sync_copy(dvb, dinv.at[pl.ds(out0, RPT)])
    pltpu.sync_copy(dv2b, dinv2.at[pl.ds(out0, RPT)])


@functools.partial(
    pl.kernel,
    out_type=(_f32((NP, D)), _f32((NP, D))),
    mesh=plsc.VectorSubcoreMesh(core_axis_name="c", subcore_axis_name="s"),
    scratch_types=[
        pltpu.VMEM((NCH2, EC2), jnp.int32),
        pltpu.VMEM((NCH2, EC2), jnp.int32),
        [pltpu.VMEM((EC2, D), jnp.float32)] * 4,
        [pltpu.SemaphoreType.DMA] * 4,
        [pltpu.SemaphoreType.DMA] * 4,
        pltpu.VMEM_SHARED((NP, D), jnp.float32),
    ],
)
def _sc_prop(didx, sidx, z_d, z_s, zrows,
             y_d, y_s,
             idx_g, idx_sc, bufs, gsems, ssems, acc):
    tid = lax.axis_index("s")
    cid = lax.axis_index("c")
    r0 = tid * RPT

    pltpu.sync_copy(zrows, acc.at[pl.ds(r0, RPT)])

    @pl.when(cid == 0)
    def _():
        pltpu.sync_copy(sidx.at[tid], idx_g)
        pltpu.sync_copy(didx.at[tid], idx_sc)

    @pl.when(cid == 1)
    def _():
        pltpu.sync_copy(didx.at[tid], idx_g)
        pltpu.sync_copy(sidx.at[tid], idx_sc)

    plsc.subcore_barrier()

    def one_side(src):
        def start_g(c, b):
            pltpu.async_copy(src.at[idx_g.at[c]], bufs[b], gsems[b])

        def start_s(c, b):
            pltpu.async_copy(bufs[b], acc.at[idx_sc.at[c]], ssems[b],
                             add=True)

        def drain(b, sems):
            pltpu.make_async_copy(src.at[pl.ds(0, EC2)], bufs[b], sems[b]).wait()

        for c in range(3):
            start_g(c, c)

        def quad(q, carry):
            c0 = 4 * q
            for j in range(4):
                c = c0 + j
                drain(j, gsems)
                start_s(c, j)
                bn = (j + 3) % 4

                @pl.when(c > 0)
                def _():
                    drain(bn, ssems)

                start_g(c + 3, bn)
            return carry

        q4 = ((NCH2 - 3) // 4) * 4
        lax.fori_loop(0, (NCH2 - 3) // 4, quad, 0)
        for c in range(q4, NCH2):
            drain(c % 4, gsems)
            start_s(c, c % 4)
            if c + 3 < NCH2:
                drain((c + 3) % 4, ssems)
                start_g(c + 3, (c + 3) % 4)
        for c in range(NCH2 - 4, NCH2):
            drain(c % 4, ssems)

    @pl.when(cid == 0)
    def _():
        one_side(z_s)

    @pl.when(cid == 1)
    def _():
        one_side(z_d)

    plsc.subcore_barrier()

    @pl.when(cid == 0)
    def _():
        pltpu.sync_copy(acc.at[pl.ds(r0, RPT)], y_d.at[pl.ds(r0, RPT)])

    @pl.when(cid == 1)
    def _():
        pltpu.sync_copy(acc.at[pl.ds(r0, RPT)], y_s.at[pl.ds(r0, RPT)])


_mat = pl.BlockSpec((RB, D), lambda i: (i, 0))
_col = pl.BlockSpec((RB, 1), lambda i: (i, 0))
_G = NP // RB


def _prep_body(ego_d, dv_d, ego_s, dv_s, z_d, z_s):
    z_d[...] = ego_d[...] * dv_d[...]
    z_s[...] = ego_s[...] * dv_s[...]


_tc_prep = pl.pallas_call(
    _prep_body, grid=(_G,),
    in_specs=[_mat, _col, _mat, _col],
    out_specs=[_mat, _mat],
    out_shape=(_f32((NP, D)), _f32((NP, D))),
)


def _first_body(y_d, d2_d, y_s, d2_s, z_d, ys_d, z_s, ys_s):
    yv = y_d[...]
    z_d[...] = yv * d2_d[...]
    ys_d[...] = yv
    yv = y_s[...]
    z_s[...] = yv * d2_s[...]
    ys_s[...] = yv


_tc_first = pl.pallas_call(
    _first_body, grid=(_G,),
    in_specs=[_mat, _col, _mat, _col],
    out_specs=[_mat, _mat, _mat, _mat],
    out_shape=(_f32((NP, D)),) * 4,
)


def _mid_body(y_d, d2_d, ysi_d, y_s, d2_s, ysi_s, z_d, yso_d, z_s, yso_s):
    yv = y_d[...]
    z_d[...] = yv * d2_d[...]
    yso_d[...] = ysi_d[...] + yv
    yv = y_s[...]
    z_s[...] = yv * d2_s[...]
    yso_s[...] = ysi_s[...] + yv


_tc_mid = pl.pallas_call(
    _mid_body, grid=(_G,),
    in_specs=[_mat, _col, _mat, _mat, _col, _mat],
    out_specs=[_mat, _mat, _mat, _mat],
    out_shape=(_f32((NP, D)),) * 4,
)


def _final_body(y_d, dv_d, ysi_d, y_s, dv_s, ysi_s, o_d, o_s):
    o_d[...] = (ysi_d[...] + y_d[...]) * dv_d[...]
    o_s[...] = (ysi_s[...] + y_s[...]) * dv_s[...]


_tc_final = pl.pallas_call(
    _final_body, grid=(_G,),
    in_specs=[_mat, _col, _mat, _mat, _col, _mat],
    out_specs=[_mat, _mat],
    out_shape=(_f32((NP, D)), _f32((NP, D))),
)


def kernel(drug_emb, disease_emb, drug_idx, disease_idx):
    di = drug_idx.astype(jnp.int32).reshape(NT, NCH, EC)
    si = disease_idx.astype(jnp.int32).reshape(NT, NCH, EC)
    pad2 = ((0, 0), (0, NCH2 * EC2 - EPT))
    di2 = jnp.pad(drug_idx.astype(jnp.int32).reshape(NT, EPT), pad2,
                  constant_values=NP - 1).reshape(NT, NCH2, EC2)
    si2 = jnp.pad(disease_idx.astype(jnp.int32).reshape(NT, EPT), pad2,
                  constant_values=NP - 1).reshape(NT, NCH2, EC2)
    ego_d = jnp.pad(drug_emb.astype(jnp.float32), ((0, NP - N_SIDE), (0, 0)))
    ego_s = jnp.pad(disease_emb.astype(jnp.float32), ((0, NP - N_SIDE), (0, 0)))

    ones1 = jnp.ones((EC,), jnp.float32)
    zeros1 = jnp.zeros((RPT,), jnp.float32)
    zrows = jnp.zeros((RPT, D), jnp.float32)

    dinv, dinv2 = _sc_stats(jnp.stack([di, si]), ones1, zeros1)
    dv_d = dinv[:NP].reshape(NP, 1)
    dv_s = dinv[NP:].reshape(NP, 1)
    d2_d = dinv2[:NP].reshape(NP, 1)
    d2_s = dinv2[NP:].reshape(NP, 1)

    z_d, z_s = _tc_prep(ego_d, dv_d, ego_s, dv_s)

    y_d, y_s = _sc_prop(di2, si2, z_d, z_s, zrows)
    z_d, ys_d, z_s, ys_s = _tc_first(y_d, d2_d, y_s, d2_s)

    y_d, y_s = _sc_prop(di2, si2, z_d, z_s, zrows)
    z_d, ys_d, z_s, ys_s = _tc_mid(y_d, d2_d, ys_d, y_s, d2_s, ys_s)

    y_d, y_s = _sc_prop(di2, si2, z_d, z_s, zrows)
    out_d, out_s = _tc_final(y_d, dv_d, ys_d, y_s, dv_s, ys_s)

    return out_d[:N_SIDE], out_s[:N_SIDE]


# speedup vs baseline: 1.3310x; 1.3310x over previous
import functools

import jax
import jax.numpy as jnp
from jax import lax
from jax.experimental import pallas as pl
from jax.experimental.pallas import tpu as pltpu
from jax.experimental.pallas import tpu_sc as plsc

N_SIDE = 5000
NP = 6144
D = 128
E = 160000
NT = 16
EPT = E // NT
EC = 80
NCH = EPT // EC
EC2 = 112
NCH2 = -(-EPT // EC2)
RPT = NP // NT
RB = 768


def _f32(shape):
    return jax.ShapeDtypeStruct(shape, jnp.float32)


@functools.partial(
    pl.kernel,
    out_type=(_f32((2 * NP,)), _f32((2 * NP,))),
    mesh=plsc.VectorSubcoreMesh(core_axis_name="c", subcore_axis_name="s"),
    scratch_types=[
        pltpu.VMEM((NCH, EC), jnp.int32),
        pltpu.VMEM((EC,), jnp.float32),
        pltpu.VMEM((RPT,), jnp.float32),
        pltpu.VMEM((RPT,), jnp.float32),
        pltpu.VMEM((RPT,), jnp.float32),
        pltpu.VMEM_SHARED((NP,), jnp.float32),
    ],
)
def _sc_stats(idx_all, ones1, zeros1,
              dinv, dinv2,
              idx_v, ones_v, degbuf, dvb, dv2b, degacc):
    tid = lax.axis_index("s")
    cid = lax.axis_index("c")
    r0 = tid * RPT

    pltpu.sync_copy(zeros1, degbuf)
    pltpu.sync_copy(degbuf, degacc.at[pl.ds(r0, RPT)])
    pltpu.sync_copy(ones1, ones_v)
    pltpu.sync_copy(idx_all.at[cid, tid], idx_v)

    plsc.subcore_barrier()

    def scat(c, carry):
        pltpu.sync_copy(ones_v, degacc.at[idx_v.at[c]], add=True)
        return carry

    lax.fori_loop(0, NCH, scat, 0)

    plsc.subcore_barrier()

    pltpu.sync_copy(degacc.at[pl.ds(r0, RPT)], degbuf)

    def conv(k, carry):
        deg = degbuf[pl.ds(k * 16, 16)]
        bits = lax.bitcast_convert_type(deg, jnp.int32)
        bits = jnp.int32(0x5F3759DF) - (bits >> 1)
        y = lax.bitcast_convert_type(bits, jnp.float32)
        for _ in range(3):
            y = y * (1.5 - 0.5 * deg * y * y)
        y = jnp.where(deg >= 0.5, y, 0.0)
        dvb[pl.ds(k * 16, 16)] = y
        dv2b[pl.ds(k * 16, 16)] = y * y
        return carry

    lax.fori_loop(0, RPT // 16, conv, 0)

    out0 = cid * NP + r0
    pltpu.sync_copy(dvb, dinv.at[pl.ds(out0, RPT)])
    pltpu.sync_copy(dv2b, dinv2.at[pl.ds(out0, RPT)])


@functools.partial(
    pl.kernel,
    out_type=(_f32((NP, D)), _f32((NP, D))),
    mesh=plsc.VectorSubcoreMesh(core_axis_name="c", subcore_axis_name="s"),
    scratch_types=[
        pltpu.VMEM((NCH2, EC2), jnp.int32),
        pltpu.VMEM((NCH2, EC2), jnp.int32),
        [pltpu.VMEM((EC2, D), jnp.float32)] * 4,
        [pltpu.SemaphoreType.DMA] * 4,
        [pltpu.SemaphoreType.DMA] * 4,
        pltpu.VMEM_SHARED((NP, D), jnp.float32),
    ],
)
def _sc_prop(didx, sidx, z_d, z_s, zrows,
             y_d, y_s,
             idx_g, idx_sc, bufs, gsems, ssems, acc):
    tid = lax.axis_index("s")
    cid = lax.axis_index("c")
    r0 = tid * RPT

    pltpu.sync_copy(zrows, acc.at[pl.ds(r0, RPT)])

    @pl.when(cid == 0)
    def _():
        pltpu.sync_copy(sidx.at[tid], idx_g)
        pltpu.sync_copy(didx.at[tid], idx_sc)

    @pl.when(cid == 1)
    def _():
        pltpu.sync_copy(didx.at[tid], idx_g)
        pltpu.sync_copy(sidx.at[tid], idx_sc)

    plsc.subcore_barrier()

    def one_side(src):
        def start_g(c, b):
            pltpu.async_copy(src.at[idx_g.at[c]], bufs[b], gsems[b])

        def start_s(c, b):
            pltpu.async_copy(bufs[b], acc.at[idx_sc.at[c]], ssems[b],
                             add=True)

        def drain(b, sems):
            pltpu.make_async_copy(src.at[pl.ds(0, EC2)], bufs[b], sems[b]).wait()

        for c in range(3):
            start_g(c, c)

        def quad(q, carry):
            c0 = 4 * q
            for j in range(4):
                c = c0 + j
                drain(j, gsems)
                start_s(c, j)
                bn = (j + 3) % 4

                @pl.when(c > 0)
                def _():
                    drain(bn, ssems)

                start_g(c + 3, bn)
            return carry

        q4 = ((NCH2 - 3) // 4) * 4
        lax.fori_loop(0, (NCH2 - 3) // 4, quad, 0)
        for c in range(q4, NCH2):
            drain(c % 4, gsems)
            start_s(c, c % 4)
            if c + 3 < NCH2:
                drain((c + 3) % 4, ssems)
                start_g(c + 3, (c + 3) % 4)
        for c in range(NCH2 - 4, NCH2):
            drain(c % 4, ssems)

    @pl.when(cid == 0)
    def _():
        one_side(z_s)

    @pl.when(cid == 1)
    def _():
        one_side(z_d)

    plsc.subcore_barrier()

    @pl.when(cid == 0)
    def _():
        pltpu.sync_copy(acc.at[pl.ds(r0, RPT)], y_d.at[pl.ds(r0, RPT)])

    @pl.when(cid == 1)
    def _():
        pltpu.sync_copy(acc.at[pl.ds(r0, RPT)], y_s.at[pl.ds(r0, RPT)])


_mat = pl.BlockSpec((RB, D), lambda i: (i, 0))
_col = pl.BlockSpec((RB, 1), lambda i: (i, 0))
_G = NP // RB


def _prep_body(ego_d, dv_d, ego_s, dv_s, z_d, z_s):
    z_d[...] = ego_d[...] * dv_d[...]
    z_s[...] = ego_s[...] * dv_s[...]


_tc_prep = pl.pallas_call(
    _prep_body, grid=(_G,),
    in_specs=[_mat, _col, _mat, _col],
    out_specs=[_mat, _mat],
    out_shape=(_f32((NP, D)), _f32((NP, D))),
)


def _first_body(y_d, d2_d, y_s, d2_s, z_d, ys_d, z_s, ys_s):
    yv = y_d[...]
    z_d[...] = yv * d2_d[...]
    ys_d[...] = yv
    yv = y_s[...]
    z_s[...] = yv * d2_s[...]
    ys_s[...] = yv


_tc_first = pl.pallas_call(
    _first_body, grid=(_G,),
    in_specs=[_mat, _col, _mat, _col],
    out_specs=[_mat, _mat, _mat, _mat],
    out_shape=(_f32((NP, D)),) * 4,
)


def _mid_body(y_d, d2_d, ysi_d, y_s, d2_s, ysi_s, z_d, yso_d, z_s, yso_s):
    yv = y_d[...]
    z_d[...] = yv * d2_d[...]
    yso_d[...] = ysi_d[...] + yv
    yv = y_s[...]
    z_s[...] = yv * d2_s[...]
    yso_s[...] = ysi_s[...] + yv


_tc_mid = pl.pallas_call(
    _mid_body, grid=(_G,),
    in_specs=[_mat, _col, _mat, _mat, _col, _mat],
    out_specs=[_mat, _mat, _mat, _mat],
    out_shape=(_f32((NP, D)),) * 4,
)


def _final_body(y_d, dv_d, ysi_d, y_s, dv_s, ysi_s, o_d, o_s):
    o_d[...] = (ysi_d[...] + y_d[...]) * dv_d[...]
    o_s[...] = (ysi_s[...] + y_s[...]) * dv_s[...]


_tc_final = pl.pallas_call(
    _final_body, grid=(_G,),
    in_specs=[_mat, _col, _mat, _mat, _col, _mat],
    out_specs=[_mat, _mat],
    out_shape=(_f32((NP, D)), _f32((NP, D))),
)


def kernel(drug_emb, disease_emb, drug_idx, disease_idx):
    di = drug_idx.astype(jnp.int32).reshape(NT, NCH, EC)
    si = disease_idx.astype(jnp.int32).reshape(NT, NCH, EC)
    pad_n = NCH2 * EC2 - EPT
    pad_rows = jnp.broadcast_to(
        N_SIDE + jnp.arange(pad_n, dtype=jnp.int32) % (NP - N_SIDE),
        (NT, pad_n))
    di2 = jnp.concatenate(
        [drug_idx.astype(jnp.int32).reshape(NT, EPT), pad_rows],
        axis=1).reshape(NT, NCH2, EC2)
    si2 = jnp.concatenate(
        [disease_idx.astype(jnp.int32).reshape(NT, EPT), pad_rows],
        axis=1).reshape(NT, NCH2, EC2)
    ego_d = jnp.pad(drug_emb.astype(jnp.float32), ((0, NP - N_SIDE), (0, 0)))
    ego_s = jnp.pad(disease_emb.astype(jnp.float32), ((0, NP - N_SIDE), (0, 0)))

    ones1 = jnp.ones((EC,), jnp.float32)
    zeros1 = jnp.zeros((RPT,), jnp.float32)
    zrows = jnp.zeros((RPT, D), jnp.float32)

    dinv, dinv2 = _sc_stats(jnp.stack([di, si]), ones1, zeros1)
    dv_d = dinv[:NP].reshape(NP, 1)
    dv_s = dinv[NP:].reshape(NP, 1)
    d2_d = dinv2[:NP].reshape(NP, 1)
    d2_s = dinv2[NP:].reshape(NP, 1)

    z_d, z_s = _tc_prep(ego_d, dv_d, ego_s, dv_s)

    y_d, y_s = _sc_prop(di2, si2, z_d, z_s, zrows)
    z_d, ys_d, z_s, ys_s = _tc_first(y_d, d2_d, y_s, d2_s)

    y_d, y_s = _sc_prop(di2, si2, z_d, z_s, zrows)
    z_d, ys_d, z_s, ys_s = _tc_mid(y_d, d2_d, ys_d, y_s, d2_s, ys_s)

    y_d, y_s = _sc_prop(di2, si2, z_d, z_s, zrows)
    out_d, out_s = _tc_final(y_d, dv_d, ys_d, y_s, dv_s, ys_s)

    return out_d[:N_SIDE], out_s[:N_SIDE]


# speedup vs baseline: 1.3575x; 1.0199x over previous
import functools

import jax
import jax.numpy as jnp
from jax import lax
from jax.experimental import pallas as pl
from jax.experimental.pallas import tpu as pltpu
from jax.experimental.pallas import tpu_sc as plsc

N_SIDE = 5000
NP = 6144
D = 128
E = 160000
NT = 16
EPT = E // NT
EC = 80
NCH = EPT // EC
EC2 = 112
NCH2 = -(-EPT // EC2)
RPT = NP // NT
RB = 768


def _f32(shape):
    return jax.ShapeDtypeStruct(shape, jnp.float32)


@functools.partial(
    pl.kernel,
    out_type=(_f32((2 * NP,)), _f32((2 * NP,))),
    mesh=plsc.VectorSubcoreMesh(core_axis_name="c", subcore_axis_name="s"),
    scratch_types=[
        pltpu.VMEM((NCH, EC), jnp.int32),
        pltpu.VMEM((EC,), jnp.float32),
        pltpu.VMEM((RPT,), jnp.float32),
        pltpu.VMEM((RPT,), jnp.float32),
        pltpu.VMEM((RPT,), jnp.float32),
        [pltpu.SemaphoreType.DMA] * 8,
        pltpu.VMEM_SHARED((NP,), jnp.float32),
    ],
)
def _sc_stats(idx_all, ones1, zeros1,
              dinv, dinv2,
              idx_v, ones_v, degbuf, dvb, dv2b, ssems, degacc):
    tid = lax.axis_index("s")
    cid = lax.axis_index("c")
    r0 = tid * RPT

    pltpu.sync_copy(zeros1, degbuf)
    pltpu.sync_copy(degbuf, degacc.at[pl.ds(r0, RPT)])
    pltpu.sync_copy(ones1, ones_v)
    pltpu.sync_copy(idx_all.at[cid, tid], idx_v)

    plsc.subcore_barrier()

    def sdrain(b):
        pltpu.make_async_copy(ones_v, degacc.at[pl.ds(0, EC)],
                              ssems[b]).wait()

    def scat8(q, carry):
        for j in range(8):
            @pl.when(q > 0)
            def _():
                sdrain(j)

            pltpu.async_copy(ones_v, degacc.at[idx_v.at[q * 8 + j]],
                             ssems[j], add=True)
        return carry

    lax.fori_loop(0, NCH // 8, scat8, 0)
    for c in range(NCH - NCH % 8, NCH):
        sdrain(c % 8)
        pltpu.async_copy(ones_v, degacc.at[idx_v.at[c]], ssems[c % 8],
                         add=True)
    for b in range(8):
        sdrain(b)

    plsc.subcore_barrier()

    pltpu.sync_copy(degacc.at[pl.ds(r0, RPT)], degbuf)

    def conv(k, carry):
        deg = degbuf[pl.ds(k * 16, 16)]
        bits = lax.bitcast_convert_type(deg, jnp.int32)
        bits = jnp.int32(0x5F3759DF) - (bits >> 1)
        y = lax.bitcast_convert_type(bits, jnp.float32)
        for _ in range(3):
            y = y * (1.5 - 0.5 * deg * y * y)
        y = jnp.where(deg >= 0.5, y, 0.0)
        dvb[pl.ds(k * 16, 16)] = y
        dv2b[pl.ds(k * 16, 16)] = y * y
        return carry

    lax.fori_loop(0, RPT // 16, conv, 0)

    out0 = cid * NP + r0
    pltpu.sync_copy(dvb, dinv.at[pl.ds(out0, RPT)])
    pltpu.sync_copy(dv2b, dinv2.at[pl.ds(out0, RPT)])


@functools.partial(
    pl.kernel,
    out_type=(_f32((NP, D)), _f32((NP, D))),
    mesh=plsc.VectorSubcoreMesh(core_axis_name="c", subcore_axis_name="s"),
    scratch_types=[
        pltpu.VMEM((NCH2, EC2), jnp.int32),
        pltpu.VMEM((NCH2, EC2), jnp.int32),
        [pltpu.VMEM((EC2, D), jnp.float32)] * 4,
        [pltpu.SemaphoreType.DMA] * 4,
        [pltpu.SemaphoreType.DMA] * 4,
        pltpu.VMEM_SHARED((NP, D), jnp.float32),
    ],
)
def _sc_prop(didx, sidx, z_d, z_s, zrows,
             y_d, y_s,
             idx_g, idx_sc, bufs, gsems, ssems, acc):
    tid = lax.axis_index("s")
    cid = lax.axis_index("c")
    r0 = tid * RPT

    pltpu.sync_copy(zrows, acc.at[pl.ds(r0, RPT)])

    @pl.when(cid == 0)
    def _():
        pltpu.sync_copy(sidx.at[tid], idx_g)
        pltpu.sync_copy(didx.at[tid], idx_sc)

    @pl.when(cid == 1)
    def _():
        pltpu.sync_copy(didx.at[tid], idx_g)
        pltpu.sync_copy(sidx.at[tid], idx_sc)

    plsc.subcore_barrier()

    def one_side(src):
        def start_g(c, b):
            pltpu.async_copy(src.at[idx_g.at[c]], bufs[b], gsems[b])

        def start_s(c, b):
            pltpu.async_copy(bufs[b], acc.at[idx_sc.at[c]], ssems[b],
                             add=True)

        def drain(b, sems):
            pltpu.make_async_copy(src.at[pl.ds(0, EC2)], bufs[b], sems[b]).wait()

        for c in range(3):
            start_g(c, c)

        def quad(q, carry):
            c0 = 4 * q
            for j in range(4):
                c = c0 + j
                drain(j, gsems)
                start_s(c, j)
                bn = (j + 3) % 4

                @pl.when(c > 0)
                def _():
                    drain(bn, ssems)

                start_g(c + 3, bn)
            return carry

        q4 = ((NCH2 - 3) // 4) * 4
        lax.fori_loop(0, (NCH2 - 3) // 4, quad, 0)
        for c in range(q4, NCH2):
            drain(c % 4, gsems)
            start_s(c, c % 4)
            if c + 3 < NCH2:
                drain((c + 3) % 4, ssems)
                start_g(c + 3, (c + 3) % 4)
        for c in range(NCH2 - 4, NCH2):
            drain(c % 4, ssems)

    @pl.when(cid == 0)
    def _():
        one_side(z_s)

    @pl.when(cid == 1)
    def _():
        one_side(z_d)

    plsc.subcore_barrier()

    @pl.when(cid == 0)
    def _():
        pltpu.sync_copy(acc.at[pl.ds(r0, RPT)], y_d.at[pl.ds(r0, RPT)])

    @pl.when(cid == 1)
    def _():
        pltpu.sync_copy(acc.at[pl.ds(r0, RPT)], y_s.at[pl.ds(r0, RPT)])


_mat = pl.BlockSpec((RB, D), lambda i: (i, 0))
_col = pl.BlockSpec((RB, 1), lambda i: (i, 0))
_G = NP // RB


def _prep_body(ego_d, dv_d, ego_s, dv_s, z_d, z_s):
    z_d[...] = ego_d[...] * dv_d[...]
    z_s[...] = ego_s[...] * dv_s[...]


_tc_prep = pl.pallas_call(
    _prep_body, grid=(_G,),
    in_specs=[_mat, _col, _mat, _col],
    out_specs=[_mat, _mat],
    out_shape=(_f32((NP, D)), _f32((NP, D))),
)


def _first_body(y_d, d2_d, y_s, d2_s, z_d, ys_d, z_s, ys_s):
    yv = y_d[...]
    z_d[...] = yv * d2_d[...]
    ys_d[...] = yv
    yv = y_s[...]
    z_s[...] = yv * d2_s[...]
    ys_s[...] = yv


_tc_first = pl.pallas_call(
    _first_body, grid=(_G,),
    in_specs=[_mat, _col, _mat, _col],
    out_specs=[_mat, _mat, _mat, _mat],
    out_shape=(_f32((NP, D)),) * 4,
)


def _mid_body(y_d, d2_d, ysi_d, y_s, d2_s, ysi_s, z_d, yso_d, z_s, yso_s):
    yv = y_d[...]
    z_d[...] = yv * d2_d[...]
    yso_d[...] = ysi_d[...] + yv
    yv = y_s[...]
    z_s[...] = yv * d2_s[...]
    yso_s[...] = ysi_s[...] + yv


_tc_mid = pl.pallas_call(
    _mid_body, grid=(_G,),
    in_specs=[_mat, _col, _mat, _mat, _col, _mat],
    out_specs=[_mat, _mat, _mat, _mat],
    out_shape=(_f32((NP, D)),) * 4,
)


def _final_body(y_d, dv_d, ysi_d, y_s, dv_s, ysi_s, o_d, o_s):
    o_d[...] = (ysi_d[...] + y_d[...]) * dv_d[...]
    o_s[...] = (ysi_s[...] + y_s[...]) * dv_s[...]


_tc_final = pl.pallas_call(
    _final_body, grid=(_G,),
    in_specs=[_mat, _col, _mat, _mat, _col, _mat],
    out_specs=[_mat, _mat],
    out_shape=(_f32((NP, D)), _f32((NP, D))),
)


def kernel(drug_emb, disease_emb, drug_idx, disease_idx):
    di = drug_idx.astype(jnp.int32).reshape(NT, NCH, EC)
    si = disease_idx.astype(jnp.int32).reshape(NT, NCH, EC)
    pad_n = NCH2 * EC2 - EPT
    pad_rows = jnp.broadcast_to(
        N_SIDE + jnp.arange(pad_n, dtype=jnp.int32) % (NP - N_SIDE),
        (NT, pad_n))
    di2 = jnp.concatenate(
        [drug_idx.astype(jnp.int32).reshape(NT, EPT), pad_rows],
        axis=1).reshape(NT, NCH2, EC2)
    si2 = jnp.concatenate(
        [disease_idx.astype(jnp.int32).reshape(NT, EPT), pad_rows],
        axis=1).reshape(NT, NCH2, EC2)
    ego_d = jnp.pad(drug_emb.astype(jnp.float32), ((0, NP - N_SIDE), (0, 0)))
    ego_s = jnp.pad(disease_emb.astype(jnp.float32), ((0, NP - N_SIDE), (0, 0)))

    ones1 = jnp.ones((EC,), jnp.float32)
    zeros1 = jnp.zeros((RPT,), jnp.float32)
    zrows = jnp.zeros((RPT, D), jnp.float32)

    dinv, dinv2 = _sc_stats(jnp.stack([di, si]), ones1, zeros1)
    dv_d = dinv[:NP].reshape(NP, 1)
    dv_s = dinv[NP:].reshape(NP, 1)
    d2_d = dinv2[:NP].reshape(NP, 1)
    d2_s = dinv2[NP:].reshape(NP, 1)

    z_d, z_s = _tc_prep(ego_d, dv_d, ego_s, dv_s)

    y_d, y_s = _sc_prop(di2, si2, z_d, z_s, zrows)
    z_d, ys_d, z_s, ys_s = _tc_first(y_d, d2_d, y_s, d2_s)

    y_d, y_s = _sc_prop(di2, si2, z_d, z_s, zrows)
    z_d, ys_d, z_s, ys_s = _tc_mid(y_d, d2_d, ys_d, y_s, d2_s, ys_s)

    y_d, y_s = _sc_prop(di2, si2, z_d, z_s, zrows)
    out_d, out_s = _tc_final(y_d, dv_d, ys_d, y_s, dv_s, ys_s)

    return out_d[:N_SIDE], out_s[:N_SIDE]
